# baseline (speedup 1.0000x reference)
import functools

import jax
import jax.numpy as jnp
from jax.experimental import pallas as pl
from jax.experimental.pallas import tpu as pltpu
from jax.experimental.pallas import tpu_sc as plsc

DIM = 64
B = 16384

_W_ENT = 256
_W_REL = 128


def _fuse_kernel(rel_ref, nrm_ref, w_ref):
    n = nrm_ref[...]
    nn = n / jnp.maximum(
        jnp.sqrt(jnp.sum(n * n, axis=-1, keepdims=True)), 1e-12)
    w_ref[...] = jnp.concatenate([rel_ref[...], nn], axis=-1)


def _fused_rel_table(rel_emb, norm_vec):
    r = rel_emb.shape[0]
    return pl.pallas_call(
        _fuse_kernel,
        out_shape=jax.ShapeDtypeStruct((r, 2 * DIM), jnp.float32),
    )(rel_emb, norm_vec)


def _sc_gather(ent_pairs, w_table, idx_ent2, idx_rel):
    n_ent = idx_ent2.shape[1]
    n_rel = idx_rel.shape[1]
    lanes = 2 * DIM
    mesh = plsc.VectorSubcoreMesh(core_axis_name="c", subcore_axis_name="s")

    @functools.partial(
        pl.kernel,
        out_type=(
            jax.ShapeDtypeStruct((n_ent, lanes), jnp.float32),
            jax.ShapeDtypeStruct((n_rel, lanes), jnp.float32),
        ),
        mesh=mesh,
    )
    def sc_kernel(ent_hbm, w_hbm, ie_hbm, ir_hbm, ent_o, rel_o):
        def ent_body(i_vmem, o_vmem):
            pltpu.sync_copy(ent_hbm.at[i_vmem.at[0]], o_vmem)

        pltpu.emit_pipeline(
            ent_body,
            grid=(n_ent // _W_ENT,),
            in_specs=[pl.BlockSpec((1, _W_ENT), lambda i: (0, i))],
            out_specs=[pl.BlockSpec((_W_ENT, lanes), lambda i: (i, 0))],
            core_axis_name=("c", "s"),
            dimension_semantics=(pltpu.PARALLEL,),
        )(ie_hbm, ent_o)

        def rel_body(i_vmem, o_vmem):
            pltpu.sync_copy(w_hbm.at[i_vmem.at[0]], o_vmem)

        pltpu.emit_pipeline(
            rel_body,
            grid=(n_rel // _W_REL,),
            in_specs=[pl.BlockSpec((1, _W_REL), lambda i: (0, i))],
            out_specs=[pl.BlockSpec((_W_REL, lanes), lambda i: (i, 0))],
            core_axis_name=("c", "s"),
            dimension_semantics=(pltpu.PARALLEL,),
        )(ir_hbm, rel_o)

    return sc_kernel(ent_pairs, w_table, idx_ent2, idx_rel)


_BLK = 2048


def _tc_loss_kernel(ph, pt, nh, nt, pw, nw,
                    ps, pts, ns, nts, out_ref):
    def pick(full, s):
        return jnp.where(s > 0.5, full[:, DIM:], full[:, :DIM])

    def score(h2, hs, t2, ts, w):
        r = w[:, :DIM]
        n = w[:, DIM:]
        h = pick(h2, hs)
        t = pick(t2, ts)

        def transfer(e):
            return e - jnp.sum(e * n, axis=-1, keepdims=True) * n

        d = transfer(h) + r - transfer(t)
        return jnp.sqrt(jnp.sum(d * d, axis=-1))

    p_score = score(ph[...], ps[...], pt[...], pts[...], pw[...])
    n_score = score(nh[...], ns[...], nt[...], nts[...], nw[...])
    partial = jnp.sum(jnp.maximum(0.0, p_score - n_score + 1.0))

    @pl.when(pl.program_id(0) == 0)
    def _():
        out_ref[0, 0] = 0.0

    out_ref[0, 0] += partial


def _tc_loss(ent_rows, w_rows, sel_ent):
    nb = B // _BLK
    full = lambda off: pl.BlockSpec((_BLK, 2 * DIM),
                                    lambda i, o=off: (i + o, 0))
    sel = lambda off: pl.BlockSpec((_BLK, 1), lambda i, o=off: (i + o, 0))
    return pl.pallas_call(
        _tc_loss_kernel,
        grid=(nb,),
        in_specs=[
            full(0), full(nb), full(2 * nb), full(3 * nb),
            full(0), full(nb),
            sel(0), sel(nb), sel(2 * nb), sel(3 * nb),
        ],
        out_specs=pl.BlockSpec(memory_space=pltpu.SMEM),
        out_shape=jax.ShapeDtypeStruct((1, 1), jnp.float32),
    )(ent_rows, ent_rows, ent_rows, ent_rows,
      w_rows, w_rows,
      sel_ent, sel_ent, sel_ent, sel_ent)


def kernel(ent_emb, rel_emb, norm_vec, pos_h, pos_r, pos_t, neg_h, neg_r, neg_t):
    ent_pairs = ent_emb.reshape(ent_emb.shape[0] // 2, 2 * DIM)
    w_table = _fused_rel_table(rel_emb, norm_vec)

    idx_ent = jnp.concatenate([pos_h, pos_t, neg_h, neg_t])
    idx_rel = jnp.concatenate([pos_r, neg_r])
    n_ent = idx_ent.shape[0]
    n_rel = idx_rel.shape[0]

    ent_rows, w_rows = _sc_gather(
        ent_pairs, w_table,
        (idx_ent // 2).reshape(1, n_ent), idx_rel.reshape(1, n_rel))

    sel_ent = (idx_ent % 2).astype(jnp.float32).reshape(n_ent, 1)
    loss = _tc_loss(ent_rows, w_rows, sel_ent)
    return loss[0, 0]


# speedup vs baseline: 1.3089x; 1.3089x over previous
import functools

import jax
import jax.numpy as jnp
from jax.experimental import pallas as pl
from jax.experimental.pallas import tpu as pltpu
from jax.experimental.pallas import tpu_sc as plsc

DIM = 64
B = 16384

_W_ENT = 256
_W_REL = 128


def _fuse_kernel(rel_ref, nrm_ref, w_ref):
    n = nrm_ref[...]
    nn = n / jnp.maximum(
        jnp.sqrt(jnp.sum(n * n, axis=-1, keepdims=True)), 1e-12)
    w_ref[...] = jnp.concatenate([rel_ref[...], nn], axis=-1)


def _fused_rel_table(rel_emb, norm_vec):
    r = rel_emb.shape[0]
    return pl.pallas_call(
        _fuse_kernel,
        out_shape=jax.ShapeDtypeStruct((r, 2 * DIM), jnp.float32),
    )(rel_emb, norm_vec)


_TCOLS = 2048


def _transpose_kernel(ent_t_ref, out_ref):
    x = ent_t_ref[...]
    out_ref[...] = jnp.concatenate(
        [x.T, jnp.zeros((_TCOLS, DIM), jnp.float32)], axis=-1)


def _transpose_pad(ent_t):
    ent = ent_t.shape[1]
    return pl.pallas_call(
        _transpose_kernel,
        grid=(pl.cdiv(ent, _TCOLS),),
        in_specs=[pl.BlockSpec((DIM, _TCOLS), lambda i: (0, i))],
        out_specs=pl.BlockSpec((_TCOLS, 2 * DIM), lambda i: (i, 0)),
        out_shape=jax.ShapeDtypeStruct((ent, 2 * DIM), jnp.float32),
    )(ent_t)


def _sc_gather(ent_pad, w_table, idx_ent, idx_rel):
    n_ent = idx_ent.shape[1]
    n_rel = idx_rel.shape[1]
    lanes = 2 * DIM
    mesh = plsc.VectorSubcoreMesh(core_axis_name="c", subcore_axis_name="s")

    @functools.partial(
        pl.kernel,
        out_type=(
            jax.ShapeDtypeStruct((n_ent, lanes), jnp.float32),
            jax.ShapeDtypeStruct((n_rel, lanes), jnp.float32),
        ),
        mesh=mesh,
    )
    def sc_kernel(ent_hbm, w_hbm, ie_hbm, ir_hbm, ent_o, rel_o):
        def ent_body(i_vmem, o_vmem):
            pltpu.sync_copy(ent_hbm.at[i_vmem.at[0]], o_vmem)

        pltpu.emit_pipeline(
            ent_body,
            grid=(n_ent // _W_ENT,),
            in_specs=[pl.BlockSpec((1, _W_ENT), lambda i: (0, i))],
            out_specs=[pl.BlockSpec((_W_ENT, lanes), lambda i: (i, 0))],
            core_axis_name=("c", "s"),
            dimension_semantics=(pltpu.PARALLEL,),
        )(ie_hbm, ent_o)

        def rel_body(i_vmem, o_vmem):
            pltpu.sync_copy(w_hbm.at[i_vmem.at[0]], o_vmem)

        pltpu.emit_pipeline(
            rel_body,
            grid=(n_rel // _W_REL,),
            in_specs=[pl.BlockSpec((1, _W_REL), lambda i: (0, i))],
            out_specs=[pl.BlockSpec((_W_REL, lanes), lambda i: (i, 0))],
            core_axis_name=("c", "s"),
            dimension_semantics=(pltpu.PARALLEL,),
        )(ir_hbm, rel_o)

    return sc_kernel(ent_pad, w_table, idx_ent, idx_rel)


_BLK = 2048


def _tc_loss_kernel(ph, pt, nh, nt, pw, nw, out_ref):
    def score(h2, t2, w):
        r = w[:, :DIM]
        n = w[:, DIM:]
        h = h2[:, :DIM]
        t = t2[:, :DIM]

        def transfer(e):
            return e - jnp.sum(e * n, axis=-1, keepdims=True) * n

        d = transfer(h) + r - transfer(t)
        return jnp.sqrt(jnp.sum(d * d, axis=-1))

    p_score = score(ph[...], pt[...], pw[...])
    n_score = score(nh[...], nt[...], nw[...])
    partial = jnp.sum(jnp.maximum(0.0, p_score - n_score + 1.0))

    @pl.when(pl.program_id(0) == 0)
    def _():
        out_ref[0, 0] = 0.0

    out_ref[0, 0] += partial


def _tc_loss(ent_rows, w_rows):
    nb = B // _BLK
    full = lambda off: pl.BlockSpec((_BLK, 2 * DIM),
                                    lambda i, o=off: (i + o, 0))
    return pl.pallas_call(
        _tc_loss_kernel,
        grid=(nb,),
        in_specs=[
            full(0), full(nb), full(2 * nb), full(3 * nb),
            full(0), full(nb),
        ],
        out_specs=pl.BlockSpec(memory_space=pltpu.SMEM),
        out_shape=jax.ShapeDtypeStruct((1, 1), jnp.float32),
    )(ent_rows, ent_rows, ent_rows, ent_rows, w_rows, w_rows)


def kernel(ent_emb, rel_emb, norm_vec, pos_h, pos_r, pos_t, neg_h, neg_r, neg_t):
    ent_pad = _transpose_pad(ent_emb.T)
    w_table = _fused_rel_table(rel_emb, norm_vec)

    idx_ent = jnp.concatenate([pos_h, pos_t, neg_h, neg_t])
    idx_rel = jnp.concatenate([pos_r, neg_r])
    n_ent = idx_ent.shape[0]
    n_rel = idx_rel.shape[0]

    ent_rows, w_rows = _sc_gather(
        ent_pad, w_table,
        idx_ent.reshape(1, n_ent), idx_rel.reshape(1, n_rel))

    loss = _tc_loss(ent_rows, w_rows)
    return loss[0, 0]


# speedup vs baseline: 1.7973x; 1.3732x over previous
import functools

import jax
import jax.numpy as jnp
from jax.experimental import pallas as pl
from jax.experimental.pallas import tpu as pltpu
from jax.experimental.pallas import tpu_sc as plsc

DIM = 64
B = 16384

_W_ENT = 256
_W_REL = 128


def _fuse_kernel(rel_ref, nrm_ref, w_ref):
    n = nrm_ref[...]
    nn = n / jnp.maximum(
        jnp.sqrt(jnp.sum(n * n, axis=-1, keepdims=True)), 1e-12)
    w_ref[...] = jnp.concatenate([rel_ref[...], nn], axis=-1)


def _fused_rel_table(rel_emb, norm_vec):
    r = rel_emb.shape[0]
    return pl.pallas_call(
        _fuse_kernel,
        out_shape=jax.ShapeDtypeStruct((r, 2 * DIM), jnp.float32),
    )(rel_emb, norm_vec)


_TCOLS = 8192
_HALF = _TCOLS // 2


def _transpose_kernel(ent_t_ref, out_ref):
    x = ent_t_ref[...]
    out_ref[...] = jnp.concatenate(
        [x[:, :_HALF].T, x[:, _HALF:].T], axis=-1)


def _transpose_pair(ent_t):
    ent = ent_t.shape[1]
    nblk = pl.cdiv(ent, _TCOLS)
    return pl.pallas_call(
        _transpose_kernel,
        grid=(nblk,),
        in_specs=[pl.BlockSpec((DIM, _TCOLS), lambda i: (0, i))],
        out_specs=pl.BlockSpec((_HALF, 2 * DIM), lambda i: (i, 0)),
        out_shape=jax.ShapeDtypeStruct((nblk * _HALF, 2 * DIM), jnp.float32),
    )(ent_t)


def _sc_gather(ent_pad, w_table, idx_ent, idx_rel):
    n_ent = idx_ent.shape[1]
    n_rel = idx_rel.shape[1]
    lanes = 2 * DIM
    mesh = plsc.VectorSubcoreMesh(core_axis_name="c", subcore_axis_name="s")

    @functools.partial(
        pl.kernel,
        out_type=(
            jax.ShapeDtypeStruct((n_ent, lanes), jnp.float32),
            jax.ShapeDtypeStruct((n_rel, lanes), jnp.float32),
        ),
        mesh=mesh,
    )
    def sc_kernel(ent_hbm, w_hbm, ie_hbm, ir_hbm, ent_o, rel_o):
        def ent_body(i_vmem, o_vmem):
            pltpu.sync_copy(ent_hbm.at[i_vmem.at[0]], o_vmem)

        pltpu.emit_pipeline(
            ent_body,
            grid=(n_ent // _W_ENT,),
            in_specs=[pl.BlockSpec((1, _W_ENT), lambda i: (0, i))],
            out_specs=[pl.BlockSpec((_W_ENT, lanes), lambda i: (i, 0))],
            core_axis_name=("c", "s"),
            dimension_semantics=(pltpu.PARALLEL,),
        )(ie_hbm, ent_o)

        def rel_body(i_vmem, o_vmem):
            pltpu.sync_copy(w_hbm.at[i_vmem.at[0]], o_vmem)

        pltpu.emit_pipeline(
            rel_body,
            grid=(n_rel // _W_REL,),
            in_specs=[pl.BlockSpec((1, _W_REL), lambda i: (0, i))],
            out_specs=[pl.BlockSpec((_W_REL, lanes), lambda i: (i, 0))],
            core_axis_name=("c", "s"),
            dimension_semantics=(pltpu.PARALLEL,),
        )(ir_hbm, rel_o)

    return sc_kernel(ent_pad, w_table, idx_ent, idx_rel)


_BLK = 2048


def _tc_loss_kernel(ph, pt, nh, nt, pw, nw,
                    ps, pts, ns, nts, out_ref):
    def pick(full, s):
        return jnp.where(s > 0.5, full[:, DIM:], full[:, :DIM])

    def score(h2, hs, t2, ts, w):
        r = w[:, :DIM]
        n = w[:, DIM:]
        h = pick(h2, hs)
        t = pick(t2, ts)

        def transfer(e):
            return e - jnp.sum(e * n, axis=-1, keepdims=True) * n

        d = transfer(h) + r - transfer(t)
        return jnp.sqrt(jnp.sum(d * d, axis=-1))

    p_score = score(ph[...], ps[...], pt[...], pts[...], pw[...])
    n_score = score(nh[...], ns[...], nt[...], nts[...], nw[...])
    partial = jnp.sum(jnp.maximum(0.0, p_score - n_score + 1.0))

    @pl.when(pl.program_id(0) == 0)
    def _():
        out_ref[0, 0] = 0.0

    out_ref[0, 0] += partial


def _tc_loss(ent_rows, w_rows, sel_ent):
    nb = B // _BLK
    full = lambda off: pl.BlockSpec((_BLK, 2 * DIM),
                                    lambda i, o=off: (i + o, 0))
    sel = lambda off: pl.BlockSpec((_BLK, 1), lambda i, o=off: (i + o, 0))
    return pl.pallas_call(
        _tc_loss_kernel,
        grid=(nb,),
        in_specs=[
            full(0), full(nb), full(2 * nb), full(3 * nb),
            full(0), full(nb),
            sel(0), sel(nb), sel(2 * nb), sel(3 * nb),
        ],
        out_specs=pl.BlockSpec(memory_space=pltpu.SMEM),
        out_shape=jax.ShapeDtypeStruct((1, 1), jnp.float32),
    )(ent_rows, ent_rows, ent_rows, ent_rows, w_rows, w_rows,
      sel_ent, sel_ent, sel_ent, sel_ent)


def kernel(ent_emb, rel_emb, norm_vec, pos_h, pos_r, pos_t, neg_h, neg_r, neg_t):
    ent_pairs = _transpose_pair(ent_emb.T)
    w_table = _fused_rel_table(rel_emb, norm_vec)

    idx_ent = jnp.concatenate([pos_h, pos_t, neg_h, neg_t])
    idx_rel = jnp.concatenate([pos_r, neg_r])
    n_ent = idx_ent.shape[0]
    n_rel = idx_rel.shape[0]

    row_ent = (idx_ent // _TCOLS) * _HALF + idx_ent % _HALF
    ent_rows, w_rows = _sc_gather(
        ent_pairs, w_table,
        row_ent.reshape(1, n_ent), idx_rel.reshape(1, n_rel))

    sel_ent = ((idx_ent // _HALF) % 2).astype(jnp.float32).reshape(n_ent, 1)
    loss = _tc_loss(ent_rows, w_rows, sel_ent)
    return loss[0, 0]


# speedup vs baseline: 1.9522x; 1.0862x over previous
import functools

import jax
import jax.numpy as jnp
from jax.experimental import pallas as pl
from jax.experimental.pallas import tpu as pltpu
from jax.experimental.pallas import tpu_sc as plsc

DIM = 64
B = 16384

_W_ENT = 256
_W_REL = 128


def _fuse_kernel(rel_ref, nrm_ref, w_ref):
    n = nrm_ref[...]
    nn = n / jnp.maximum(
        jnp.sqrt(jnp.sum(n * n, axis=-1, keepdims=True)), 1e-12)
    w_ref[...] = jnp.concatenate([rel_ref[...], nn], axis=-1)


def _fused_rel_table(rel_emb, norm_vec):
    r = rel_emb.shape[0]
    return pl.pallas_call(
        _fuse_kernel,
        out_shape=jax.ShapeDtypeStruct((r, 2 * DIM), jnp.float32),
    )(rel_emb, norm_vec)


_TCOLS = 16384
_HALF = _TCOLS // 2


def _transpose_kernel(ent_t_ref, out_ref):
    x = ent_t_ref[...]
    out_ref[...] = jnp.concatenate(
        [x[:, :_HALF].T, x[:, _HALF:].T], axis=-1)


def _transpose_pair(ent_t):
    ent = ent_t.shape[1]
    nblk = pl.cdiv(ent, _TCOLS)
    return pl.pallas_call(
        _transpose_kernel,
        grid=(nblk,),
        in_specs=[pl.BlockSpec((DIM, _TCOLS), lambda i: (0, i))],
        out_specs=pl.BlockSpec((_HALF, 2 * DIM), lambda i: (i, 0)),
        out_shape=jax.ShapeDtypeStruct((nblk * _HALF, 2 * DIM), jnp.float32),
    )(ent_t)


def _sc_gather(ent_pad, w_table, idx_ent, idx_rel):
    n_ent = idx_ent.shape[1]
    n_rel = idx_rel.shape[1]
    lanes = 2 * DIM
    mesh = plsc.VectorSubcoreMesh(core_axis_name="c", subcore_axis_name="s")

    @functools.partial(
        pl.kernel,
        out_type=(
            jax.ShapeDtypeStruct((n_ent, lanes), jnp.float32),
            jax.ShapeDtypeStruct((n_rel, lanes), jnp.float32),
        ),
        mesh=mesh,
    )
    def sc_kernel(ent_hbm, w_hbm, ie_hbm, ir_hbm, ent_o, rel_o):
        def ent_body(i_vmem, o_vmem):
            pltpu.sync_copy(ent_hbm.at[i_vmem.at[0]], o_vmem)

        pltpu.emit_pipeline(
            ent_body,
            grid=(n_ent // _W_ENT,),
            in_specs=[pl.BlockSpec((1, _W_ENT), lambda i: (0, i))],
            out_specs=[pl.BlockSpec((_W_ENT, lanes), lambda i: (i, 0))],
            core_axis_name=("c", "s"),
            dimension_semantics=(pltpu.PARALLEL,),
        )(ie_hbm, ent_o)

        def rel_body(i_vmem, o_vmem):
            pltpu.sync_copy(w_hbm.at[i_vmem.at[0]], o_vmem)

        pltpu.emit_pipeline(
            rel_body,
            grid=(n_rel // _W_REL,),
            in_specs=[pl.BlockSpec((1, _W_REL), lambda i: (0, i))],
            out_specs=[pl.BlockSpec((_W_REL, lanes), lambda i: (i, 0))],
            core_axis_name=("c", "s"),
            dimension_semantics=(pltpu.PARALLEL,),
        )(ir_hbm, rel_o)

    return sc_kernel(ent_pad, w_table, idx_ent, idx_rel)


_BLK = 2048


def _tc_loss_kernel(ph, pt, nh, nt, pw, nw,
                    ps, pts, ns, nts, out_ref):
    def pick(full, s):
        return jnp.where(s > 0.5, full[:, DIM:], full[:, :DIM])

    def score(h2, hs, t2, ts, w):
        r = w[:, :DIM]
        n = w[:, DIM:]
        h = pick(h2, hs)
        t = pick(t2, ts)

        def transfer(e):
            return e - jnp.sum(e * n, axis=-1, keepdims=True) * n

        d = transfer(h) + r - transfer(t)
        return jnp.sqrt(jnp.sum(d * d, axis=-1))

    p_score = score(ph[...], ps[...], pt[...], pts[...], pw[...])
    n_score = score(nh[...], ns[...], nt[...], nts[...], nw[...])
    partial = jnp.sum(jnp.maximum(0.0, p_score - n_score + 1.0))

    @pl.when(pl.program_id(0) == 0)
    def _():
        out_ref[0, 0] = 0.0

    out_ref[0, 0] += partial


def _tc_loss(ent_rows, w_rows, sel_ent):
    nb = B // _BLK
    full = lambda off: pl.BlockSpec((_BLK, 2 * DIM),
                                    lambda i, o=off: (i + o, 0))
    sel = lambda off: pl.BlockSpec((_BLK, 1), lambda i, o=off: (i + o, 0))
    return pl.pallas_call(
        _tc_loss_kernel,
        grid=(nb,),
        in_specs=[
            full(0), full(nb), full(2 * nb), full(3 * nb),
            full(0), full(nb),
            sel(0), sel(nb), sel(2 * nb), sel(3 * nb),
        ],
        out_specs=pl.BlockSpec(memory_space=pltpu.SMEM),
        out_shape=jax.ShapeDtypeStruct((1, 1), jnp.float32),
    )(ent_rows, ent_rows, ent_rows, ent_rows, w_rows, w_rows,
      sel_ent, sel_ent, sel_ent, sel_ent)


def kernel(ent_emb, rel_emb, norm_vec, pos_h, pos_r, pos_t, neg_h, neg_r, neg_t):
    ent_pairs = _transpose_pair(ent_emb.T)
    w_table = _fused_rel_table(rel_emb, norm_vec)

    idx_ent = jnp.concatenate([pos_h, pos_t, neg_h, neg_t])
    idx_rel = jnp.concatenate([pos_r, neg_r])
    n_ent = idx_ent.shape[0]
    n_rel = idx_rel.shape[0]

    row_ent = (idx_ent // _TCOLS) * _HALF + idx_ent % _HALF
    ent_rows, w_rows = _sc_gather(
        ent_pairs, w_table,
        row_ent.reshape(1, n_ent), idx_rel.reshape(1, n_rel))

    sel_ent = ((idx_ent // _HALF) % 2).astype(jnp.float32).reshape(n_ent, 1)
    loss = _tc_loss(ent_rows, w_rows, sel_ent)
    return loss[0, 0]


# speedup vs baseline: 2.0383x; 1.0441x over previous
import functools

import jax
import jax.numpy as jnp
from jax.experimental import pallas as pl
from jax.experimental.pallas import tpu as pltpu
from jax.experimental.pallas import tpu_sc as plsc

DIM = 64
B = 16384

_W_ENT = 256
_W_REL = 128


def _fuse_kernel(rel_ref, nrm_ref, w_ref):
    n = nrm_ref[...]
    nn = n / jnp.maximum(
        jnp.sqrt(jnp.sum(n * n, axis=-1, keepdims=True)), 1e-12)
    w_ref[...] = jnp.concatenate([rel_ref[...], nn], axis=-1)


def _fused_rel_table(rel_emb, norm_vec):
    r = rel_emb.shape[0]
    return pl.pallas_call(
        _fuse_kernel,
        out_shape=jax.ShapeDtypeStruct((r, 2 * DIM), jnp.float32),
    )(rel_emb, norm_vec)


_TCOLS = 32768
_HALF = _TCOLS // 2


def _transpose_kernel(ent_t_ref, out_ref):
    x = ent_t_ref[...]
    out_ref[...] = jnp.concatenate(
        [x[:, :_HALF].T, x[:, _HALF:].T], axis=-1)


def _transpose_pair(ent_t):
    ent = ent_t.shape[1]
    nblk = pl.cdiv(ent, _TCOLS)
    return pl.pallas_call(
        _transpose_kernel,
        grid=(nblk,),
        in_specs=[pl.BlockSpec((DIM, _TCOLS), lambda i: (0, i))],
        out_specs=pl.BlockSpec((_HALF, 2 * DIM), lambda i: (i, 0)),
        out_shape=jax.ShapeDtypeStruct((nblk * _HALF, 2 * DIM), jnp.float32),
    )(ent_t)


def _sc_gather(ent_pad, w_table, idx_ent, idx_rel):
    n_ent = idx_ent.shape[1]
    n_rel = idx_rel.shape[1]
    lanes = 2 * DIM
    mesh = plsc.VectorSubcoreMesh(core_axis_name="c", subcore_axis_name="s")

    @functools.partial(
        pl.kernel,
        out_type=(
            jax.ShapeDtypeStruct((n_ent, lanes), jnp.float32),
            jax.ShapeDtypeStruct((n_rel, lanes), jnp.float32),
        ),
        mesh=mesh,
    )
    def sc_kernel(ent_hbm, w_hbm, ie_hbm, ir_hbm, ent_o, rel_o):
        def ent_body(i_vmem, o_vmem):
            pltpu.sync_copy(ent_hbm.at[i_vmem.at[0]], o_vmem)

        pltpu.emit_pipeline(
            ent_body,
            grid=(n_ent // _W_ENT,),
            in_specs=[pl.BlockSpec((1, _W_ENT), lambda i: (0, i))],
            out_specs=[pl.BlockSpec((_W_ENT, lanes), lambda i: (i, 0))],
            core_axis_name=("c", "s"),
            dimension_semantics=(pltpu.PARALLEL,),
        )(ie_hbm, ent_o)

        def rel_body(i_vmem, o_vmem):
            pltpu.sync_copy(w_hbm.at[i_vmem.at[0]], o_vmem)

        pltpu.emit_pipeline(
            rel_body,
            grid=(n_rel // _W_REL,),
            in_specs=[pl.BlockSpec((1, _W_REL), lambda i: (0, i))],
            out_specs=[pl.BlockSpec((_W_REL, lanes), lambda i: (i, 0))],
            core_axis_name=("c", "s"),
            dimension_semantics=(pltpu.PARALLEL,),
        )(ir_hbm, rel_o)

    return sc_kernel(ent_pad, w_table, idx_ent, idx_rel)


_BLK = 2048


def _tc_loss_kernel(ph, pt, nh, nt, pw, nw,
                    ps, pts, ns, nts, out_ref):
    def pick(full, s):
        return jnp.where(s > 0.5, full[:, DIM:], full[:, :DIM])

    def score(h2, hs, t2, ts, w):
        r = w[:, :DIM]
        n = w[:, DIM:]
        h = pick(h2, hs)
        t = pick(t2, ts)

        def transfer(e):
            return e - jnp.sum(e * n, axis=-1, keepdims=True) * n

        d = transfer(h) + r - transfer(t)
        return jnp.sqrt(jnp.sum(d * d, axis=-1))

    p_score = score(ph[...], ps[...], pt[...], pts[...], pw[...])
    n_score = score(nh[...], ns[...], nt[...], nts[...], nw[...])
    partial = jnp.sum(jnp.maximum(0.0, p_score - n_score + 1.0))

    @pl.when(pl.program_id(0) == 0)
    def _():
        out_ref[0, 0] = 0.0

    out_ref[0, 0] += partial


def _tc_loss(ent_rows, w_rows, sel_ent):
    nb = B // _BLK
    full = lambda off: pl.BlockSpec((_BLK, 2 * DIM),
                                    lambda i, o=off: (i + o, 0))
    sel = lambda off: pl.BlockSpec((_BLK, 1), lambda i, o=off: (i + o, 0))
    return pl.pallas_call(
        _tc_loss_kernel,
        grid=(nb,),
        in_specs=[
            full(0), full(nb), full(2 * nb), full(3 * nb),
            full(0), full(nb),
            sel(0), sel(nb), sel(2 * nb), sel(3 * nb),
        ],
        out_specs=pl.BlockSpec(memory_space=pltpu.SMEM),
        out_shape=jax.ShapeDtypeStruct((1, 1), jnp.float32),
    )(ent_rows, ent_rows, ent_rows, ent_rows, w_rows, w_rows,
      sel_ent, sel_ent, sel_ent, sel_ent)


def kernel(ent_emb, rel_emb, norm_vec, pos_h, pos_r, pos_t, neg_h, neg_r, neg_t):
    ent_pairs = _transpose_pair(ent_emb.T)
    w_table = _fused_rel_table(rel_emb, norm_vec)

    idx_ent = jnp.concatenate([pos_h, pos_t, neg_h, neg_t])
    idx_rel = jnp.concatenate([pos_r, neg_r])
    n_ent = idx_ent.shape[0]
    n_rel = idx_rel.shape[0]

    row_ent = (idx_ent // _TCOLS) * _HALF + idx_ent % _HALF
    ent_rows, w_rows = _sc_gather(
        ent_pairs, w_table,
        row_ent.reshape(1, n_ent), idx_rel.reshape(1, n_rel))

    sel_ent = ((idx_ent // _HALF) % 2).astype(jnp.float32).reshape(n_ent, 1)
    loss = _tc_loss(ent_rows, w_rows, sel_ent)
    return loss[0, 0]


# speedup vs baseline: 2.0487x; 1.0051x over previous
import functools

import jax
import jax.numpy as jnp
from jax.experimental import pallas as pl
from jax.experimental.pallas import tpu as pltpu
from jax.experimental.pallas import tpu_sc as plsc

DIM = 64
B = 16384

_W_ENT = 256
_W_REL = 128


def _fuse_kernel(rel_ref, nrm_ref, w_ref):
    n = nrm_ref[...]
    nn = n / jnp.maximum(
        jnp.sqrt(jnp.sum(n * n, axis=-1, keepdims=True)), 1e-12)
    w_ref[...] = jnp.concatenate([rel_ref[...], nn], axis=-1)


def _fused_rel_table(rel_emb, norm_vec):
    r = rel_emb.shape[0]
    return pl.pallas_call(
        _fuse_kernel,
        out_shape=jax.ShapeDtypeStruct((r, 2 * DIM), jnp.float32),
    )(rel_emb, norm_vec)


_TCOLS = 32768
_HALF = _TCOLS // 2


def _transpose_kernel(ent_t_ref, out_ref):
    x = ent_t_ref[...]
    out_ref[...] = jnp.concatenate(
        [x[:, :_HALF].T, x[:, _HALF:].T], axis=-1)


def _transpose_pair(ent_t):
    ent = ent_t.shape[1]
    nblk = pl.cdiv(ent, _TCOLS)
    return pl.pallas_call(
        _transpose_kernel,
        grid=(nblk,),
        in_specs=[pl.BlockSpec((DIM, _TCOLS), lambda i: (0, i))],
        out_specs=pl.BlockSpec((_HALF, 2 * DIM), lambda i: (i, 0)),
        out_shape=jax.ShapeDtypeStruct((nblk * _HALF, 2 * DIM), jnp.float32),
    )(ent_t)


def _sc_gather_rows(table, idx, window):
    n = idx.shape[1]
    lanes = 2 * DIM
    mesh = plsc.VectorSubcoreMesh(core_axis_name="c", subcore_axis_name="s")

    @functools.partial(
        pl.kernel,
        out_type=jax.ShapeDtypeStruct((n, lanes), jnp.float32),
        mesh=mesh,
    )
    def sc_kernel(t_hbm, i_hbm, o_hbm):
        def body(i_vmem, o_vmem):
            pltpu.sync_copy(t_hbm.at[i_vmem.at[0]], o_vmem)

        pltpu.emit_pipeline(
            body,
            grid=(n // window,),
            in_specs=[pl.BlockSpec((1, window), lambda i: (0, i))],
            out_specs=[pl.BlockSpec((window, lanes), lambda i: (i, 0))],
            core_axis_name=("c", "s"),
            dimension_semantics=(pltpu.PARALLEL,),
        )(i_hbm, o_hbm)

    return sc_kernel(table, idx)


_BLK = 2048


def _tc_loss_kernel(ph, pt, nh, nt, pw, nw,
                    ps, pts, ns, nts, out_ref):
    def pick(full, s):
        return jnp.where(s > 0.5, full[:, DIM:], full[:, :DIM])

    def score(h2, hs, t2, ts, w):
        r = w[:, :DIM]
        n = w[:, DIM:]
        h = pick(h2, hs)
        t = pick(t2, ts)

        def transfer(e):
            return e - jnp.sum(e * n, axis=-1, keepdims=True) * n

        d = transfer(h) + r - transfer(t)
        return jnp.sqrt(jnp.sum(d * d, axis=-1))

    p_score = score(ph[...], ps[...], pt[...], pts[...], pw[...])
    n_score = score(nh[...], ns[...], nt[...], nts[...], nw[...])
    partial = jnp.sum(jnp.maximum(0.0, p_score - n_score + 1.0))

    @pl.when(pl.program_id(0) == 0)
    def _():
        out_ref[0, 0] = 0.0

    out_ref[0, 0] += partial


def _tc_loss(ent_rows, w_rows, sel_ent):
    nb = B // _BLK
    full = lambda off: pl.BlockSpec((_BLK, 2 * DIM),
                                    lambda i, o=off: (i + o, 0))
    sel = lambda off: pl.BlockSpec((_BLK, 1), lambda i, o=off: (i + o, 0))
    return pl.pallas_call(
        _tc_loss_kernel,
        grid=(nb,),
        in_specs=[
            full(0), full(nb), full(2 * nb), full(3 * nb),
            full(0), full(nb),
            sel(0), sel(nb), sel(2 * nb), sel(3 * nb),
        ],
        out_specs=pl.BlockSpec(memory_space=pltpu.SMEM),
        out_shape=jax.ShapeDtypeStruct((1, 1), jnp.float32),
    )(ent_rows, ent_rows, ent_rows, ent_rows, w_rows, w_rows,
      sel_ent, sel_ent, sel_ent, sel_ent)


def kernel(ent_emb, rel_emb, norm_vec, pos_h, pos_r, pos_t, neg_h, neg_r, neg_t):
    w_table = _fused_rel_table(rel_emb, norm_vec)
    idx_rel = jnp.concatenate([pos_r, neg_r])
    n_rel = idx_rel.shape[0]
    w_rows = _sc_gather_rows(w_table, idx_rel.reshape(1, n_rel), _W_REL)

    ent_pairs = _transpose_pair(ent_emb.T)
    idx_ent = jnp.concatenate([pos_h, pos_t, neg_h, neg_t])
    n_ent = idx_ent.shape[0]
    row_ent = (idx_ent // _TCOLS) * _HALF + idx_ent % _HALF
    ent_rows = _sc_gather_rows(ent_pairs, row_ent.reshape(1, n_ent), _W_ENT)

    sel_ent = ((idx_ent // _HALF) % 2).astype(jnp.float32).reshape(n_ent, 1)
    loss = _tc_loss(ent_rows, w_rows, sel_ent)
    return loss[0, 0]


# speedup vs baseline: 2.1604x; 1.0545x over previous
import functools

import jax
import jax.numpy as jnp
from jax.experimental import pallas as pl
from jax.experimental.pallas import tpu as pltpu
from jax.experimental.pallas import tpu_sc as plsc

DIM = 64
B = 16384

_W_ENT = 256
_W_REL = 128


def _fuse_kernel(rel_ref, nrm_ref, w_ref):
    n = nrm_ref[...]
    nn = n / jnp.maximum(
        jnp.sqrt(jnp.sum(n * n, axis=-1, keepdims=True)), 1e-12)
    w_ref[...] = jnp.concatenate([rel_ref[...], nn], axis=-1)


def _fused_rel_table(rel_emb, norm_vec):
    r = rel_emb.shape[0]
    return pl.pallas_call(
        _fuse_kernel,
        out_shape=jax.ShapeDtypeStruct((r, 2 * DIM), jnp.float32),
    )(rel_emb, norm_vec)


_TCOLS = 32768
_HALF = _TCOLS // 2


def _transpose_kernel(ent_t_ref, out_ref):
    x = ent_t_ref[...]
    out_ref[...] = jnp.concatenate(
        [x[:, :_HALF].T, x[:, _HALF:].T], axis=-1)


def _transpose_pair(ent_t):
    ent = ent_t.shape[1]
    nblk = pl.cdiv(ent, _TCOLS)
    return pl.pallas_call(
        _transpose_kernel,
        grid=(nblk,),
        in_specs=[pl.BlockSpec((DIM, _TCOLS), lambda i: (0, i))],
        out_specs=pl.BlockSpec((_HALF, 2 * DIM), lambda i: (i, 0)),
        out_shape=jax.ShapeDtypeStruct((nblk * _HALF, 2 * DIM), jnp.float32),
    )(ent_t)


def _sc_gather_rows(table, idx, window):
    n = idx.shape[1]
    lanes = 2 * DIM
    mesh = plsc.VectorSubcoreMesh(core_axis_name="c", subcore_axis_name="s")

    @functools.partial(
        pl.kernel,
        out_type=jax.ShapeDtypeStruct((n, lanes), jnp.float32),
        mesh=mesh,
    )
    def sc_kernel(t_hbm, i_hbm, o_hbm):
        def body(i_vmem, o_vmem):
            pltpu.sync_copy(t_hbm.at[i_vmem.at[0]], o_vmem)

        pltpu.emit_pipeline(
            body,
            grid=(n // window,),
            in_specs=[pl.BlockSpec((1, window), lambda i: (0, i))],
            out_specs=[pl.BlockSpec((window, lanes), lambda i: (i, 0))],
            core_axis_name=("c", "s"),
            dimension_semantics=(pltpu.PARALLEL,),
        )(i_hbm, o_hbm)

    return sc_kernel(table, idx)


_BLK = 2048


def _tc_loss_kernel(ph, pt, nh, nt, pw, nw,
                    ps, pts, ns, nts, out_ref):
    def pick(full, s8):
        s = s8[...].T[:, 0:1]
        return jnp.where(s > 0.5, full[:, DIM:], full[:, :DIM])

    def score(h2, hs, t2, ts, w):
        r = w[:, :DIM]
        n = w[:, DIM:]
        h = pick(h2, hs)
        t = pick(t2, ts)

        def transfer(e):
            return e - jnp.sum(e * n, axis=-1, keepdims=True) * n

        d = transfer(h) + r - transfer(t)
        return jnp.sqrt(jnp.sum(d * d, axis=-1))

    p_score = score(ph[...], ps[...], pt[...], pts[...], pw[...])
    n_score = score(nh[...], ns[...], nt[...], nts[...], nw[...])
    partial = jnp.sum(jnp.maximum(0.0, p_score - n_score + 1.0))

    @pl.when(pl.program_id(0) == 0)
    def _():
        out_ref[0, 0] = 0.0

    out_ref[0, 0] += partial


def _tc_loss(ent_rows, w_rows, sel_ent):
    nb = B // _BLK
    full = lambda off: pl.BlockSpec((_BLK, 2 * DIM),
                                    lambda i, o=off: (i + o, 0))
    sel = lambda off: pl.BlockSpec((8, _BLK), lambda i, o=off: (0, i + o))
    return pl.pallas_call(
        _tc_loss_kernel,
        grid=(nb,),
        in_specs=[
            full(0), full(nb), full(2 * nb), full(3 * nb),
            full(0), full(nb),
            sel(0), sel(nb), sel(2 * nb), sel(3 * nb),
        ],
        out_specs=pl.BlockSpec(memory_space=pltpu.SMEM),
        out_shape=jax.ShapeDtypeStruct((1, 1), jnp.float32),
    )(ent_rows, ent_rows, ent_rows, ent_rows, w_rows, w_rows,
      sel_ent, sel_ent, sel_ent, sel_ent)


def kernel(ent_emb, rel_emb, norm_vec, pos_h, pos_r, pos_t, neg_h, neg_r, neg_t):
    w_table = _fused_rel_table(rel_emb, norm_vec)
    idx_rel = jnp.concatenate([pos_r, neg_r])
    n_rel = idx_rel.shape[0]
    w_rows = _sc_gather_rows(w_table, idx_rel.reshape(1, n_rel), _W_REL)

    ent_pairs = _transpose_pair(ent_emb.T)
    idx_ent = jnp.concatenate([pos_h, pos_t, neg_h, neg_t])
    n_ent = idx_ent.shape[0]
    row_ent = (idx_ent // _TCOLS) * _HALF + idx_ent % _HALF
    ent_rows = _sc_gather_rows(ent_pairs, row_ent.reshape(1, n_ent), _W_ENT)

    sel_ent = jnp.broadcast_to(
        ((idx_ent // _HALF) % 2).astype(jnp.float32)[None, :], (8, n_ent))
    loss = _tc_loss(ent_rows, w_rows, sel_ent)
    return loss[0, 0]
